# Initial kernel scaffold; baseline (speedup 1.0000x reference)
import jax
import jax.numpy as jnp
from jax.experimental import pallas as pl


def kernel(newSample, xyz1, neighborsMatrix, numNeighbors, weightMatrix, eigC, eigV, eigVT):
    raise NotImplementedError("write your pallas kernel here")


# speedup vs baseline: 61.1454x; 61.1454x over previous
import functools

import jax
import jax.numpy as jnp
from jax import lax
from jax.experimental import pallas as pl
from jax.experimental.pallas import tpu as pltpu
from jax.experimental.pallas import tpu_sc as plsc

_N = 50000
_K = 32
_NC = 128
_3N = 3 * _N

_CBA = 15360


def _deform_body(eigv_ref, delta_ref, xyz_ref, out_ref):
    disp = jnp.dot(eigv_ref[...], delta_ref[...],
                   preferred_element_type=jnp.float32)
    out_ref[...] = disp + xyz_ref[...]


def _deformed_flat(eigV, delta, xyz_flat):
    grid = pl.cdiv(_3N, _CBA)
    return pl.pallas_call(
        _deform_body,
        grid=(grid,),
        in_specs=[
            pl.BlockSpec((_CBA, _NC), lambda i: (i, 0)),
            pl.BlockSpec((_NC, 1), lambda i: (0, 0)),
            pl.BlockSpec((_CBA, 1), lambda i: (i, 0)),
        ],
        out_specs=pl.BlockSpec((_CBA, 1), lambda i: (i, 0)),
        out_shape=jax.ShapeDtypeStruct((_3N, 1), jnp.float32),
    )(eigV, delta, xyz_flat)


_TOT = _K * _N
_NW = 32
_BPW = _TOT // _NW
_CH = 1000
_NIT = _BPW // _CH
_TD = 128


def _sc_gather(table, idx):
    mesh = plsc.VectorSubcoreMesh(core_axis_name="c", subcore_axis_name="s")

    @functools.partial(
        pl.kernel,
        mesh=mesh,
        out_type=jax.ShapeDtypeStruct((_TOT, _TD), jnp.float32),
        scratch_types=[
            pltpu.VMEM((_CH,), jnp.int32),
            pltpu.VMEM((_CH, _TD), jnp.float32),
            pltpu.SemaphoreType.DMA,
        ],
    )
    def gather_kernel(table_hbm, idx_hbm, out_hbm, idx_v, rows_v, sem):
        wid = lax.axis_index("s") * 2 + lax.axis_index("c")
        base = wid * _BPW

        @pl.loop(0, _NIT)
        def _chunk(i):
            off = base + i * _CH
            pltpu.sync_copy(idx_hbm.at[pl.ds(off, _CH)], idx_v)
            pltpu.async_copy(table_hbm.at[idx_v], rows_v, sem).wait()
            pltpu.sync_copy(rows_v, out_hbm.at[pl.ds(off, _CH)])

    return gather_kernel(table, idx)


_VB = 512


def _jacobi_rot(A, V, p, q):
    apq = A[p][q]
    app = A[p][p]
    aqq = A[q][q]
    small = jnp.abs(apq) < 1e-30
    tau = (aqq - app) / jnp.where(small, 1.0, 2.0 * apq)
    sgn = jnp.where(tau >= 0.0, 1.0, -1.0)
    t = sgn / (jnp.abs(tau) + jnp.sqrt(1.0 + tau * tau))
    t = jnp.where(small, 0.0, t)
    c = 1.0 / jnp.sqrt(1.0 + t * t)
    s = t * c
    M = [row[:] for row in A]
    for r in range(3):
        mp = c * A[r][p] - s * A[r][q]
        mq = s * A[r][p] + c * A[r][q]
        M[r][p] = mp
        M[r][q] = mq
    A2 = [row[:] for row in M]
    for cc in range(3):
        ap = c * M[p][cc] - s * M[q][cc]
        aq = s * M[p][cc] + c * M[q][cc]
        A2[p][cc] = ap
        A2[q][cc] = aq
    V2 = [row[:] for row in V]
    for r in range(3):
        vp = c * V[r][p] - s * V[r][q]
        vq = s * V[r][p] + c * V[r][q]
        V2[r][p] = vp
        V2[r][q] = vq
    return A2, V2


def _arap_body(gat_ref, wt_ref, xyzT_ref, defT_ref, msg_ref, gsum_ref,
               esum_ref):
    vb = pl.program_id(0)
    w = wt_ref[...]

    cx = xyzT_ref[0:1, :]
    cy = xyzT_ref[1:2, :]
    cz = xyzT_ref[2:3, :]
    dxx = defT_ref[0:1, :]
    dyy = defT_ref[1:2, :]
    dzz = defT_ref[2:3, :]

    e1 = [cx - gat_ref[:, :, 0], cy - gat_ref[:, :, 1], cz - gat_ref[:, :, 2]]
    e2 = [dxx - gat_ref[:, :, 3], dyy - gat_ref[:, :, 4],
          dzz - gat_ref[:, :, 5]]

    S = [[jnp.sum(w * e1[i] * e2[j], axis=0, keepdims=True)
          for j in range(3)] for i in range(3)]
    A = [[S[0][i] * S[0][j] + S[1][i] * S[1][j] + S[2][i] * S[2][j]
          for j in range(3)] for i in range(3)]
    one = jnp.ones_like(A[0][0])
    zero = jnp.zeros_like(A[0][0])
    V = [[one, zero, zero], [zero, one, zero], [zero, zero, one]]
    for _ in range(6):
        for (p, q) in ((0, 1), (0, 2), (1, 2)):
            A, V = _jacobi_rot(A, V, p, q)
    lam = [A[0][0], A[1][1], A[2][2]]
    vc = [[V[r][i] for r in range(3)] for i in range(3)]
    for (i, j) in ((0, 1), (1, 2), (0, 1)):
        cnd = lam[i] < lam[j]
        lam[i], lam[j] = (jnp.where(cnd, lam[j], lam[i]),
                          jnp.where(cnd, lam[i], lam[j]))
        for r in range(3):
            a_, b_ = vc[i][r], vc[j][r]
            vc[i][r] = jnp.where(cnd, b_, a_)
            vc[j][r] = jnp.where(cnd, a_, b_)

    def matv(M3, x):
        return [M3[r][0] * x[0] + M3[r][1] * x[1] + M3[r][2] * x[2]
                for r in range(3)]

    def norm3(x):
        return jnp.sqrt(x[0] * x[0] + x[1] * x[1] + x[2] * x[2])

    def cross3(a, b):
        return [a[1] * b[2] - a[2] * b[1],
                a[2] * b[0] - a[0] * b[2],
                a[0] * b[1] - a[1] * b[0]]

    tiny = 1e-30
    sv1 = matv(S, vc[0])
    n1 = norm3(sv1)
    ok1 = n1 > tiny
    u1 = [jnp.where(ok1, sv1[r] / jnp.where(ok1, n1, 1.0),
                    (one if r == 0 else zero)) for r in range(3)]
    sv2 = matv(S, vc[1])
    dot12 = u1[0] * sv2[0] + u1[1] * sv2[1] + u1[2] * sv2[2]
    t2 = [sv2[r] - dot12 * u1[r] for r in range(3)]
    n2 = norm3(t2)
    f1 = [zero, u1[2], -u1[1]]
    f2 = [-u1[2], zero, u1[0]]
    use1 = (u1[1] * u1[1] + u1[2] * u1[2]) > 0.01
    fb = [jnp.where(use1, f1[r], f2[r]) for r in range(3)]
    nfb = norm3(fb)
    fb = [fb[r] / jnp.where(nfb > tiny, nfb, 1.0) for r in range(3)]
    rank2 = n2 > 1e-4 * (n1 + tiny)
    u2 = [jnp.where(rank2, t2[r] / jnp.where(rank2, n2, 1.0), fb[r])
          for r in range(3)]
    u3 = cross3(u1, u2)
    cv = cross3(vc[1], vc[2])
    detv = vc[0][0] * cv[0] + vc[0][1] * cv[1] + vc[0][2] * cv[2]
    R = [[vc[0][r] * u1[c] + vc[1][r] * u2[c] + detv * vc[2][r] * u3[c]
          for c in range(3)] for r in range(3)]

    rot = [R[r][0] * e1[0] + R[r][1] * e1[1] + R[r][2] * e1[2]
           for r in range(3)]
    D = [e2[r] - rot[r] for r in range(3)]
    msg = [w * D[r] for r in range(3)]

    lane = lax.broadcasted_iota(jnp.int32, (1, _VB), 1)
    valid = lane < (_N - vb * _VB)
    e_pv = jnp.sum(w * (D[0] * D[0] + D[1] * D[1] + D[2] * D[2]),
                   axis=0, keepdims=True)
    e_pv = jnp.where(valid, e_pv, 0.0)

    for r in range(3):
        msg_ref[r, :, :] = msg[r]
        gsum_ref[r:r + 1, :] = jnp.sum(msg[r], axis=0, keepdims=True)

    @pl.when(vb == 0)
    def _init():
        esum_ref[...] = jnp.zeros_like(esum_ref)

    esum_ref[...] += jnp.sum(e_pv, axis=1, keepdims=True)


def _arap_blocks(gat, wT, xyzT, defT):
    grid = pl.cdiv(_N, _VB)
    return pl.pallas_call(
        _arap_body,
        grid=(grid,),
        in_specs=[
            pl.BlockSpec((_K, _VB, _TD), lambda i: (0, i, 0)),
            pl.BlockSpec((_K, _VB), lambda i: (0, i)),
            pl.BlockSpec((3, _VB), lambda i: (0, i)),
            pl.BlockSpec((3, _VB), lambda i: (0, i)),
        ],
        out_specs=[
            pl.BlockSpec((3, _K, _VB), lambda i: (0, 0, i)),
            pl.BlockSpec((3, _VB), lambda i: (0, i)),
            pl.BlockSpec((1, 1), lambda i: (0, 0)),
        ],
        out_shape=[
            jax.ShapeDtypeStruct((3, _K, _N), jnp.float32),
            jax.ShapeDtypeStruct((3, _N), jnp.float32),
            jax.ShapeDtypeStruct((1, 1), jnp.float32),
        ],
    )(gat, wT, xyzT, defT)


def _proj_body(eigv_ref, g_ref, out_ref):
    i = pl.program_id(0)
    row = lax.broadcasted_iota(jnp.int32, (_CBA, 1), 0)
    valid = row < (_3N - i * _CBA)
    g = jnp.where(valid, g_ref[...], 0.0)
    part = jnp.sum(eigv_ref[...] * g, axis=0, keepdims=True)

    @pl.when(i == 0)
    def _init():
        out_ref[...] = jnp.zeros_like(out_ref)

    out_ref[...] += part


def _project(eigV, g_flat):
    grid = pl.cdiv(_3N, _CBA)
    return pl.pallas_call(
        _proj_body,
        grid=(grid,),
        in_specs=[
            pl.BlockSpec((_CBA, _NC), lambda i: (i, 0)),
            pl.BlockSpec((_CBA, 1), lambda i: (i, 0)),
        ],
        out_specs=pl.BlockSpec((1, _NC), lambda i: (0, 0)),
        out_shape=jax.ShapeDtypeStruct((1, _NC), jnp.float32),
    )(eigV, g_flat)


@jax.jit
def kernel(newSample, xyz1, neighborsMatrix, numNeighbors, weightMatrix,
           eigC, eigV, eigVT):
    delta = (eigC * newSample).reshape(_NC, 1)
    xyz_flat = xyz1.reshape(_3N, 1)

    deformed_flat = _deformed_flat(eigV, delta, xyz_flat)
    deformed = deformed_flat.reshape(_N, 3)

    table = jnp.concatenate(
        [xyz1, deformed, jnp.zeros((_N, _TD - 6), jnp.float32)], axis=1)
    idx_flat = neighborsMatrix.T.reshape(_TOT)
    gat = _sc_gather(table, idx_flat).reshape(_K, _N, _TD)

    mask = (jnp.arange(_K)[None, :] < numNeighbors[:, None]).astype(
        jnp.float32)
    wT = (weightMatrix * mask).T
    xyzT = xyz1.T
    defT = deformed.T

    msg, gsum, esum = _arap_blocks(gat, wT, xyzT, defT)

    energy = esum[0, 0] / _N
    scat = jax.vmap(
        lambda m: jax.ops.segment_sum(m.reshape(_TOT), idx_flat, _N))(msg)
    g = (2.0 / _N) * (gsum - scat)
    g_flat = g.T.reshape(_3N, 1)

    grad_pre = _project(eigV, g_flat).reshape(_NC)
    grad = eigC * grad_pre
    return energy, grad
